# baseline (speedup 1.0000x reference)
import functools

import jax
import jax.numpy as jnp
from jax import lax
from jax.experimental import pallas as pl
from jax.experimental.pallas import tpu as pltpu
from jax.experimental.pallas import tpu_sc as plsc

VOCAB = 1000000
EMBED = 64
B_TOTAL = 4096 * 200

_info = plsc.get_sparse_core_info()
NC, NS = _info.num_cores, _info.num_subcores
NW = NC * NS
B_PER_W = B_TOTAL // NW
GROUP = 128
N_GROUPS = B_PER_W // GROUP


@functools.partial(
    pl.kernel,
    out_type=jax.ShapeDtypeStruct((B_TOTAL, EMBED), jnp.float32),
    mesh=plsc.VectorSubcoreMesh(core_axis_name="c", subcore_axis_name="s"),
    scratch_types=[
        pltpu.VMEM((B_PER_W,), jnp.int32),
        pltpu.VMEM((GROUP, EMBED), jnp.float32),
        pltpu.SemaphoreType.DMA,
    ],
    compiler_params=pltpu.CompilerParams(use_tc_tiling_on_sc=False),
)
def _gather_kernel(idx_hbm, table_hbm, out_hbm, idx_v, rows_v, sem):
    wid = lax.axis_index("s") * NC + lax.axis_index("c")
    base = wid * B_PER_W
    pltpu.sync_copy(idx_hbm.at[pl.ds(base, B_PER_W)], idx_v)

    def body(g):
        off = g * GROUP
        pltpu.async_copy(
            table_hbm.at[idx_v.at[pl.ds(off, GROUP)]], rows_v, sem
        ).wait()
        pltpu.sync_copy(rows_v, out_hbm.at[pl.ds(base + off, GROUP)])

    pl.loop(0, N_GROUPS)(body)


def kernel(indices, wte):
    flat = indices.reshape(-1)
    out = _gather_kernel(flat, wte)
    return out.reshape(indices.shape + (EMBED,))


# speedup vs baseline: 1.1119x; 1.1119x over previous
import functools

import jax
import jax.numpy as jnp
from jax import lax
from jax.experimental import pallas as pl
from jax.experimental.pallas import tpu as pltpu
from jax.experimental.pallas import tpu_sc as plsc

VOCAB = 1000000
EMBED = 64
B_TOTAL = 4096 * 200

_info = plsc.get_sparse_core_info()
NC, NS = _info.num_cores, _info.num_subcores
NW = NC * NS
B_PER_W = B_TOTAL // NW
GROUP = 128
N_GROUPS = B_PER_W // GROUP
NBUF = 4
N_CHUNKS = N_GROUPS // NBUF


@functools.partial(
    pl.kernel,
    out_type=jax.ShapeDtypeStruct((B_TOTAL, EMBED), jnp.float32),
    mesh=plsc.VectorSubcoreMesh(core_axis_name="c", subcore_axis_name="s"),
    scratch_types=[
        pltpu.VMEM((B_PER_W,), jnp.int32),
        pltpu.VMEM((NBUF, GROUP, EMBED), jnp.float32),
        pltpu.SemaphoreType.DMA((NBUF,)),
        pltpu.SemaphoreType.DMA((NBUF,)),
    ],
    compiler_params=pltpu.CompilerParams(use_tc_tiling_on_sc=False),
)
def _gather_kernel(idx_hbm, table_hbm, out_hbm, idx_v, bufs, gsems, osems):
    wid = lax.axis_index("s") * NC + lax.axis_index("c")
    base = wid * B_PER_W
    pltpu.sync_copy(idx_hbm.at[pl.ds(base, B_PER_W)], idx_v)

    def gather(g, b):
        return pltpu.make_async_copy(
            table_hbm.at[idx_v.at[pl.ds(g * GROUP, GROUP)]],
            bufs.at[b],
            gsems.at[b],
        )

    def outcopy(g, b):
        return pltpu.make_async_copy(
            bufs.at[b],
            out_hbm.at[pl.ds(base + g * GROUP, GROUP)],
            osems.at[b],
        )

    def chunk(c):
        for b in range(NBUF):
            g = c * NBUF + b

            @pl.when(c > 0)
            def _():
                outcopy(g - NBUF, b).wait()

            gather(g, b).start()
        for b in range(NBUF):
            g = c * NBUF + b
            gather(g, b).wait()
            outcopy(g, b).start()

    pl.loop(0, N_CHUNKS)(chunk)
    for b in range(NBUF):
        outcopy((N_CHUNKS - 1) * NBUF + b, b).wait()


def kernel(indices, wte):
    flat = indices.reshape(-1)
    out = _gather_kernel(flat, wte)
    return out.reshape(indices.shape + (EMBED,))


# speedup vs baseline: 1.1143x; 1.0021x over previous
import functools

import jax
import jax.numpy as jnp
from jax import lax
from jax.experimental import pallas as pl
from jax.experimental.pallas import tpu as pltpu
from jax.experimental.pallas import tpu_sc as plsc

VOCAB = 1000000
EMBED = 64
B_TOTAL = 4096 * 200

_info = plsc.get_sparse_core_info()
NC, NS = _info.num_cores, _info.num_subcores
NW = NC * NS
B_PER_W = B_TOTAL // NW
GROUP = 256
N_GROUPS = B_PER_W // GROUP
NBUF = 4
N_CHUNKS = N_GROUPS // NBUF


@functools.partial(
    pl.kernel,
    out_type=jax.ShapeDtypeStruct((B_TOTAL, EMBED), jnp.float32),
    mesh=plsc.VectorSubcoreMesh(core_axis_name="c", subcore_axis_name="s"),
    scratch_types=[
        pltpu.VMEM((B_PER_W,), jnp.int32),
        pltpu.VMEM((NBUF, GROUP, EMBED), jnp.float32),
        pltpu.SemaphoreType.DMA((NBUF,)),
        pltpu.SemaphoreType.DMA((NBUF,)),
    ],
    compiler_params=pltpu.CompilerParams(use_tc_tiling_on_sc=False),
)
def _gather_kernel(idx_hbm, table_hbm, out_hbm, idx_v, bufs, gsems, osems):
    wid = lax.axis_index("s") * NC + lax.axis_index("c")
    base = wid * B_PER_W
    pltpu.sync_copy(idx_hbm.at[pl.ds(base, B_PER_W)], idx_v)

    def gather(g, b):
        return pltpu.make_async_copy(
            table_hbm.at[idx_v.at[pl.ds(g * GROUP, GROUP)]],
            bufs.at[b],
            gsems.at[b],
        )

    def outcopy(g, b):
        return pltpu.make_async_copy(
            bufs.at[b],
            out_hbm.at[pl.ds(base + g * GROUP, GROUP)],
            osems.at[b],
        )

    def chunk(c):
        for b in range(NBUF):
            g = c * NBUF + b

            @pl.when(c > 0)
            def _():
                outcopy(g - NBUF, b).wait()

            gather(g, b).start()
        for b in range(NBUF):
            g = c * NBUF + b
            gather(g, b).wait()
            outcopy(g, b).start()

    pl.loop(0, N_CHUNKS)(chunk)
    for b in range(NBUF):
        outcopy((N_CHUNKS - 1) * NBUF + b, b).wait()


def kernel(indices, wte):
    flat = indices.reshape(-1)
    out = _gather_kernel(flat, wte)
    return out.reshape(indices.shape + (EMBED,))
